# Initial kernel scaffold; baseline (speedup 1.0000x reference)
import functools

import jax
import jax.numpy as jnp
from jax import lax
from jax.experimental import pallas as pl
from jax.experimental.pallas import tpu as pltpu
from jax.experimental.pallas import tpu_sc as plsc

N_NODES = 10000
N_EDGES = 320000
IN_CH = 128
HID_CH = 256
OUT_CH = 128

NC = 2
NS = 16
NPAD = 10240
RPT = NPAD // NS
K = 80
EPT = N_EDGES // NS
NCH = EPT // K
EPW = N_EDGES // (NC * NS)
NCHD = EPW // K
DW = 16

_MESH = dict(core_axis_name="c", subcore_axis_name="s")


def _zero_rows(buf, nrows, ncols):
    zero16 = jnp.zeros((16,), jnp.float32)

    def row(r, carry):
        for kk in range(ncols // 16):
            buf[r, pl.ds(kk * 16, 16)] = zero16
        return carry

    lax.fori_loop(0, nrows, row, 0)


@functools.partial(
    pl.kernel,
    out_type=[
        jax.ShapeDtypeStruct((NPAD, DW), jnp.float32),
        jax.ShapeDtypeStruct((NPAD, DW), jnp.float32),
    ],
    mesh=plsc.VectorSubcoreMesh(**_MESH),
    scratch_types=[
        pltpu.VMEM((NCHD, K), jnp.int32),
        pltpu.VMEM((K, DW), jnp.float32),
        pltpu.VMEM((K, DW), jnp.float32),
        pltpu.VMEM_SHARED((NPAD, DW), jnp.float32),
    ],
)
def _deg_kernel(dstd, deg0, deg1, dst_v, ones_v, zero_v, acc):
    c = lax.axis_index("c")
    s = lax.axis_index("s")
    wid = s * NC + c
    pltpu.sync_copy(dstd.at[wid], dst_v)

    one16 = jnp.ones((16,), jnp.float32)

    def fill(r, carry):
        ones_v[r] = one16
        return carry

    lax.fori_loop(0, K, fill, 0)
    _zero_rows(zero_v, K, DW)
    for z in range(RPT // K):
        pltpu.sync_copy(zero_v, acc.at[pl.ds(s * RPT + z * K, K)])
    plsc.subcore_barrier()

    def body(j, carry):
        pltpu.sync_copy(ones_v, acc.at[dst_v.at[j]], add=True)
        return carry

    lax.fori_loop(0, NCHD, body, 0)
    plsc.subcore_barrier()

    @pl.when(c == 0)
    def _():
        pltpu.sync_copy(acc.at[pl.ds(s * RPT, RPT)], deg0.at[pl.ds(s * RPT, RPT)])

    @pl.when(c == 1)
    def _():
        pltpu.sync_copy(acc.at[pl.ds(s * RPT, RPT)], deg1.at[pl.ds(s * RPT, RPT)])


def _make_agg(C):
    @functools.partial(
        pl.kernel,
        out_type=[
            jax.ShapeDtypeStruct((NPAD, C), jnp.float32),
            jax.ShapeDtypeStruct((NPAD, C), jnp.float32),
        ],
        mesh=plsc.VectorSubcoreMesh(**_MESH),
        scratch_types=[
            pltpu.VMEM((NCH, K), jnp.int32),
            pltpu.VMEM((NCH, K), jnp.int32),
            pltpu.VMEM((K, C), jnp.float32),
            pltpu.VMEM((K, C), jnp.float32),
            pltpu.VMEM_SHARED((NPAD, C), jnp.float32),
            pltpu.SemaphoreType.DMA,
            pltpu.SemaphoreType.DMA,
        ],
    )
    def agg(y0, y1, srcr, dstr, out0, out1,
            src_v, dst_v, buf_a, buf_b, acc, sem_a, sem_b):
        c = lax.axis_index("c")
        s = lax.axis_index("s")
        pltpu.sync_copy(srcr.at[s], src_v)
        pltpu.sync_copy(dstr.at[s], dst_v)

        _zero_rows(buf_a, K, C)
        for z in range(RPT // K):
            pltpu.sync_copy(buf_a, acc.at[pl.ds(s * RPT + z * K, K)])
        plsc.subcore_barrier()

        def gather(j, buf, sem):
            @pl.when(c == 0)
            def _():
                pltpu.async_copy(y0.at[src_v.at[j]], buf, sem)

            @pl.when(c == 1)
            def _():
                pltpu.async_copy(y1.at[src_v.at[j]], buf, sem)

        def wait_gather(j, buf, sem):
            pltpu.make_async_copy(y0.at[src_v.at[j]], buf, sem).wait()

        def body(i, carry):
            j0 = 2 * i
            j1 = j0 + 1
            gather(j0, buf_a, sem_a)
            gather(j1, buf_b, sem_b)
            wait_gather(j0, buf_a, sem_a)
            pltpu.sync_copy(buf_a, acc.at[dst_v.at[j0]], add=True)
            wait_gather(j1, buf_b, sem_b)
            pltpu.sync_copy(buf_b, acc.at[dst_v.at[j1]], add=True)
            return carry

        lax.fori_loop(0, NCH // 2, body, 0)
        plsc.subcore_barrier()

        @pl.when(c == 0)
        def _():
            pltpu.sync_copy(acc.at[pl.ds(s * RPT, RPT)], out0.at[pl.ds(s * RPT, RPT)])

        @pl.when(c == 1)
        def _():
            pltpu.sync_copy(acc.at[pl.ds(s * RPT, RPT)], out1.at[pl.ds(s * RPT, RPT)])

    return agg


_agg128 = _make_agg(IN_CH)
_agg64 = _make_agg(OUT_CH // 2)


_RB = 1000
_GRID = N_NODES // _RB


def _dinv_of(d0_ref, d1_ref):
    deg = d0_ref[:, 0] + d1_ref[:, 0] + 1.0
    return lax.rsqrt(deg)


def _tc_a_body(x_ref, w1_ref, d0_ref, d1_ref, b1_ref, y0_ref, y1_ref, self_ref):
    dinv = _dinv_of(d0_ref, d1_ref)[:, None]
    xw = jnp.dot(x_ref[...], w1_ref[...], preferred_element_type=jnp.float32)
    y = xw * dinv
    y0_ref[...] = y[:, :IN_CH]
    y1_ref[...] = y[:, IN_CH:]
    self_ref[...] = y * dinv + b1_ref[...]


def _tc_a(x, W1, d0, d1, b1):
    return pl.pallas_call(
        _tc_a_body,
        grid=(_GRID,),
        in_specs=[
            pl.BlockSpec((_RB, IN_CH), lambda i: (i, 0)),
            pl.BlockSpec((IN_CH, HID_CH), lambda i: (0, 0)),
            pl.BlockSpec((_RB, DW), lambda i: (i, 0)),
            pl.BlockSpec((_RB, DW), lambda i: (i, 0)),
            pl.BlockSpec((1, HID_CH), lambda i: (0, 0)),
        ],
        out_specs=[
            pl.BlockSpec((_RB, IN_CH), lambda i: (i, 0)),
            pl.BlockSpec((_RB, IN_CH), lambda i: (i, 0)),
            pl.BlockSpec((_RB, HID_CH), lambda i: (i, 0)),
        ],
        out_shape=[
            jax.ShapeDtypeStruct((N_NODES, IN_CH), jnp.float32),
            jax.ShapeDtypeStruct((N_NODES, IN_CH), jnp.float32),
            jax.ShapeDtypeStruct((N_NODES, HID_CH), jnp.float32),
        ],
    )(x, W1, d0, d1, b1)


def _tc_b_body(a0_ref, a1_ref, self_ref, d0_ref, d1_ref, w2_ref, b2_ref,
               y20_ref, y21_ref, self2_ref):
    dinv = _dinv_of(d0_ref, d1_ref)[:, None]
    acc = jnp.concatenate([a0_ref[...], a1_ref[...]], axis=1)
    h = jnp.maximum(acc * dinv + self_ref[...], 0.0)
    xw2 = jnp.dot(h, w2_ref[...], preferred_element_type=jnp.float32)
    y2 = xw2 * dinv
    y20_ref[...] = y2[:, : OUT_CH // 2]
    y21_ref[...] = y2[:, OUT_CH // 2:]
    self2_ref[...] = y2 * dinv + b2_ref[...]


def _tc_b(a0, a1, selft, d0, d1, W2, b2):
    return pl.pallas_call(
        _tc_b_body,
        grid=(_GRID,),
        in_specs=[
            pl.BlockSpec((_RB, IN_CH), lambda i: (i, 0)),
            pl.BlockSpec((_RB, IN_CH), lambda i: (i, 0)),
            pl.BlockSpec((_RB, HID_CH), lambda i: (i, 0)),
            pl.BlockSpec((_RB, DW), lambda i: (i, 0)),
            pl.BlockSpec((_RB, DW), lambda i: (i, 0)),
            pl.BlockSpec((HID_CH, OUT_CH), lambda i: (0, 0)),
            pl.BlockSpec((1, OUT_CH), lambda i: (0, 0)),
        ],
        out_specs=[
            pl.BlockSpec((_RB, OUT_CH // 2), lambda i: (i, 0)),
            pl.BlockSpec((_RB, OUT_CH // 2), lambda i: (i, 0)),
            pl.BlockSpec((_RB, OUT_CH), lambda i: (i, 0)),
        ],
        out_shape=[
            jax.ShapeDtypeStruct((N_NODES, OUT_CH // 2), jnp.float32),
            jax.ShapeDtypeStruct((N_NODES, OUT_CH // 2), jnp.float32),
            jax.ShapeDtypeStruct((N_NODES, OUT_CH), jnp.float32),
        ],
    )(a0, a1, selft, d0, d1, W2, b2)


def _tc_c_body(a0_ref, a1_ref, self2_ref, d0_ref, d1_ref, out_ref):
    dinv = _dinv_of(d0_ref, d1_ref)[:, None]
    acc = jnp.concatenate([a0_ref[...], a1_ref[...]], axis=1)
    out_ref[...] = acc * dinv + self2_ref[...]


def _tc_c(a0, a1, self2, d0, d1):
    return pl.pallas_call(
        _tc_c_body,
        grid=(_GRID,),
        in_specs=[
            pl.BlockSpec((_RB, OUT_CH // 2), lambda i: (i, 0)),
            pl.BlockSpec((_RB, OUT_CH // 2), lambda i: (i, 0)),
            pl.BlockSpec((_RB, OUT_CH), lambda i: (i, 0)),
            pl.BlockSpec((_RB, DW), lambda i: (i, 0)),
            pl.BlockSpec((_RB, DW), lambda i: (i, 0)),
        ],
        out_specs=pl.BlockSpec((_RB, OUT_CH), lambda i: (i, 0)),
        out_shape=jax.ShapeDtypeStruct((N_NODES, OUT_CH), jnp.float32),
    )(a0, a1, self2, d0, d1)


def kernel(x, edge_index, W1, b1, W2, b2):
    ei = edge_index.astype(jnp.int32)
    src = ei[0]
    dst = ei[1]
    srcr = src.reshape(NS, NCH, K)
    dstr = dst.reshape(NS, NCH, K)
    dstd = dst.reshape(NC * NS, NCHD, K)

    deg0, deg1 = _deg_kernel(dstd)
    y0, y1, selft = _tc_a(x, W1, deg0, deg1, b1.reshape(1, HID_CH))
    a0, a1 = _agg128(y0, y1, srcr, dstr)
    y20, y21, self2 = _tc_b(a0, a1, selft, deg0, deg1, W2, b2.reshape(1, OUT_CH))
    a20, a21 = _agg64(y20, y21, srcr, dstr)
    return _tc_c(a20, a21, self2, deg0, deg1)


# speedup vs baseline: 21.0228x; 21.0228x over previous
import functools

import jax
import jax.numpy as jnp
from jax import lax
from jax.experimental import pallas as pl
from jax.experimental.pallas import tpu as pltpu
from jax.experimental.pallas import tpu_sc as plsc

N_NODES = 10000
N_EDGES = 320000
IN_CH = 128
HID_CH = 256
OUT_CH = 128

NC = 2
NS = 16
NW = NC * NS
NPAD = 10240
RPT = NPAD // NS
K = 80
EPW = N_EDGES // NW
NCHW = EPW // K
DW = 8

_MESH = dict(core_axis_name="c", subcore_axis_name="s",
             num_cores=NC, num_subcores=NS)


def _writeback(acc, out0, out1, c, s):
    @pl.when(c == 0)
    def _():
        pltpu.sync_copy(acc.at[pl.ds(s * RPT, RPT)],
                        out0.at[pl.ds(s * RPT, RPT)])

    @pl.when(c == 1)
    def _():
        pltpu.sync_copy(acc.at[pl.ds(s * RPT, RPT)],
                        out1.at[pl.ds(s * RPT, RPT)])


@functools.cache
def _make_deg():
    return functools.partial(
        pl.kernel,
        out_type=[
            jax.ShapeDtypeStruct((NPAD, DW), jnp.float32),
            jax.ShapeDtypeStruct((NPAD, DW), jnp.float32),
        ],
        mesh=plsc.VectorSubcoreMesh(**_MESH),
        scratch_types=[
            pltpu.VMEM((NCHW, K), jnp.int32),
            pltpu.VMEM((K, DW), jnp.float32),
            pltpu.VMEM((K, DW), jnp.float32),
            pltpu.VMEM_SHARED((NPAD, DW), jnp.float32),
        ],
    )(_deg_body)


def _deg_body(dstw, ones_h, zero_h, deg0, deg1, dst_v, ones_v, zero_v, acc):
    c = lax.axis_index("c")
    s = lax.axis_index("s")
    wid = s * NC + c
    pltpu.sync_copy(dstw.at[wid], dst_v)
    pltpu.sync_copy(ones_h, ones_v)
    pltpu.sync_copy(zero_h, zero_v)
    for z in range(RPT // K):
        pltpu.sync_copy(zero_v, acc.at[pl.ds(s * RPT + z * K, K)])
    plsc.subcore_barrier()

    def body(j, carry):
        pltpu.sync_copy(ones_v, acc.at[dst_v.at[j]], add=True)
        return carry

    lax.fori_loop(0, NCHW, body, 0)
    plsc.subcore_barrier()
    _writeback(acc, deg0, deg1, c, s)


@functools.cache
def _make_agg():
    @functools.partial(
        pl.kernel,
        out_type=[
            jax.ShapeDtypeStruct((NPAD, IN_CH), jnp.float32),
            jax.ShapeDtypeStruct((NPAD, IN_CH), jnp.float32),
        ],
        mesh=plsc.VectorSubcoreMesh(**_MESH),
        scratch_types=[
            pltpu.VMEM((NCHW, K), jnp.int32),
            pltpu.VMEM((NCHW, K), jnp.int32),
            pltpu.VMEM((K, IN_CH), jnp.float32),
            pltpu.VMEM_SHARED((NPAD, IN_CH), jnp.float32),
        ],
    )
    def agg(tbl, srcw, dstw, zero_h, out0, out1,
            src_v, dst_v, buf_a, acc):
        c = lax.axis_index("c")
        s = lax.axis_index("s")
        wid = s * NC + c
        pltpu.sync_copy(srcw.at[wid], src_v)
        pltpu.sync_copy(dstw.at[wid], dst_v)
        pltpu.sync_copy(zero_h, buf_a)
        for z in range(RPT // K):
            pltpu.sync_copy(buf_a, acc.at[pl.ds(s * RPT + z * K, K)])
        plsc.subcore_barrier()

        def body(j, carry):
            pltpu.sync_copy(tbl.at[src_v.at[j]], buf_a)
            pltpu.sync_copy(buf_a, acc.at[dst_v.at[j]], add=True)
            return carry

        lax.fori_loop(0, NCHW, body, 0)
        plsc.subcore_barrier()
        _writeback(acc, out0, out1, c, s)

    return agg


_RB = 1000
_GRID = N_NODES // _RB
_ROW = lambda i: (i, 0)
_ALL = lambda i: (0, 0)


def _dinv_of(d0_ref, d1_ref):
    deg = d0_ref[:, 0] + d1_ref[:, 0] + 1.0
    return lax.rsqrt(deg)


def _tc_a_body(x_ref, w1_ref, d0_ref, d1_ref, b1_ref, u_ref, self_ref):
    dinv = _dinv_of(d0_ref, d1_ref)[:, None]
    u_ref[...] = x_ref[...] * dinv
    xw = jnp.dot(x_ref[...], w1_ref[...], preferred_element_type=jnp.float32)
    self_ref[...] = xw * (dinv * dinv) + b1_ref[...]


def _tc_a(x, W1, d0, d1, b1):
    return pl.pallas_call(
        _tc_a_body,
        grid=(_GRID,),
        in_specs=[
            pl.BlockSpec((_RB, IN_CH), _ROW),
            pl.BlockSpec((IN_CH, HID_CH), _ALL),
            pl.BlockSpec((_RB, DW), _ROW),
            pl.BlockSpec((_RB, DW), _ROW),
            pl.BlockSpec((1, HID_CH), _ALL),
        ],
        out_specs=[pl.BlockSpec((_RB, IN_CH), _ROW),
                   pl.BlockSpec((_RB, HID_CH), _ROW)],
        out_shape=[jax.ShapeDtypeStruct((N_NODES, IN_CH), jnp.float32),
                   jax.ShapeDtypeStruct((N_NODES, HID_CH), jnp.float32)],
    )(x, W1, d0, d1, b1)


def _tc_b_body(p0_ref, p1_ref, self_ref, d0_ref, d1_ref, w1_ref, w2_ref,
               b2_ref, y2_ref, self2_ref):
    dinv = _dinv_of(d0_ref, d1_ref)[:, None]
    s1 = p0_ref[...] + p1_ref[...]
    agg = jnp.dot(s1, w1_ref[...], preferred_element_type=jnp.float32)
    h = jnp.maximum(agg * dinv + self_ref[...], 0.0)
    xw2 = jnp.dot(h, w2_ref[...], preferred_element_type=jnp.float32)
    y2 = xw2 * dinv
    y2_ref[...] = y2
    self2_ref[...] = y2 * dinv + b2_ref[...]


def _tc_b(p0, p1, selft, d0, d1, W1, W2, b2):
    return pl.pallas_call(
        _tc_b_body,
        grid=(_GRID,),
        in_specs=[
            pl.BlockSpec((_RB, IN_CH), _ROW),
            pl.BlockSpec((_RB, IN_CH), _ROW),
            pl.BlockSpec((_RB, HID_CH), _ROW),
            pl.BlockSpec((_RB, DW), _ROW),
            pl.BlockSpec((_RB, DW), _ROW),
            pl.BlockSpec((IN_CH, HID_CH), _ALL),
            pl.BlockSpec((HID_CH, OUT_CH), _ALL),
            pl.BlockSpec((1, OUT_CH), _ALL),
        ],
        out_specs=[pl.BlockSpec((_RB, OUT_CH), _ROW),
                   pl.BlockSpec((_RB, OUT_CH), _ROW)],
        out_shape=[jax.ShapeDtypeStruct((N_NODES, OUT_CH), jnp.float32),
                   jax.ShapeDtypeStruct((N_NODES, OUT_CH), jnp.float32)],
    )(p0, p1, selft, d0, d1, W1, W2, b2)


def _tc_c_body(q0_ref, q1_ref, self2_ref, d0_ref, d1_ref, out_ref):
    dinv = _dinv_of(d0_ref, d1_ref)[:, None]
    out_ref[...] = (q0_ref[...] + q1_ref[...]) * dinv + self2_ref[...]


def _tc_c(q0, q1, self2, d0, d1):
    return pl.pallas_call(
        _tc_c_body,
        grid=(_GRID,),
        in_specs=[
            pl.BlockSpec((_RB, OUT_CH), _ROW),
            pl.BlockSpec((_RB, OUT_CH), _ROW),
            pl.BlockSpec((_RB, OUT_CH), _ROW),
            pl.BlockSpec((_RB, DW), _ROW),
            pl.BlockSpec((_RB, DW), _ROW),
        ],
        out_specs=pl.BlockSpec((_RB, OUT_CH), _ROW),
        out_shape=jax.ShapeDtypeStruct((N_NODES, OUT_CH), jnp.float32),
    )(q0, q1, self2, d0, d1)


def kernel(x, edge_index, W1, b1, W2, b2):
    ei = edge_index.astype(jnp.int32)
    srcw = ei[0].reshape(NW, NCHW, K)
    dstw = ei[1].reshape(NW, NCHW, K)
    ones_h = jnp.ones((K, DW), jnp.float32)
    zdeg_h = jnp.zeros((K, DW), jnp.float32)
    zf32_h = jnp.zeros((K, IN_CH), jnp.float32)

    agg = _make_agg()
    deg0, deg1 = _make_deg()(dstw, ones_h, zdeg_h)
    d0 = deg0[:N_NODES]
    d1 = deg1[:N_NODES]
    u, selft = _tc_a(x, W1, d0, d1, b1.reshape(1, HID_CH))
    p0, p1 = agg(u, srcw, dstw, zf32_h)
    y2, self2 = _tc_b(p0[:N_NODES], p1[:N_NODES], selft, d0, d1, W1, W2,
                      b2.reshape(1, OUT_CH))
    q0, q1 = agg(y2, srcw, dstw, zf32_h)
    return _tc_c(q0[:N_NODES], q1[:N_NODES], self2, d0, d1)
